# baseline (speedup 1.0000x reference)
import functools

import jax
import jax.numpy as jnp
from jax.experimental import pallas as pl
from jax.experimental.pallas import tpu as pltpu

N_NODES = 512
D_FEAT = 256
N_HIDDEN = 256


def _dot(a, b):
    return jax.lax.dot_general(
        a, b, (((1,), (0,)), ((), ())),
        preferred_element_type=jnp.float32)


def _gcn_kernel(gids_ref, order_ref, x_ref, a_ref, mask_ref,
                Win_ref, bin_ref, W1_ref, b1_ref, W2_ref, b2_ref,
                W3_ref, b3_ref, out_ref, h_scratch):
    b = pl.program_id(0)
    prev = gids_ref[jnp.maximum(b - 1, 0)]
    is_new = jnp.logical_or(b == 0, gids_ref[b] != prev)

    @pl.when(is_new)
    def _compute():
        bf = jnp.bfloat16
        x = x_ref[0].astype(bf)
        a = a_ref[0].astype(bf)
        h0 = jax.nn.relu(_dot(x, Win_ref[...]) + bin_ref[...])
        h = jax.nn.relu(_dot(_dot(a, h0.astype(bf)).astype(bf), W1_ref[...]) + b1_ref[...])
        h = jax.nn.relu(_dot(_dot(a, h.astype(bf)).astype(bf), W2_ref[...]) + b2_ref[...])
        h = jnp.tanh(_dot(_dot(a, h.astype(bf)).astype(bf), W3_ref[...]) + b3_ref[...])
        h_scratch[...] = h + h0

    m = mask_ref[0]
    denom = jnp.maximum(jnp.sum(m), 1.0)
    out_ref[0] = _dot(m, h_scratch[...]) / denom


def kernel(graph, coverpoint_mask, batch_xs, batch_as, W_in, b_in,
           W1, b1, W2, b2, W3, b3):
    B = graph.shape[0]
    order = jnp.argsort(graph).astype(jnp.int32)
    gids = jnp.take(graph, order).astype(jnp.int32)
    mask_f = coverpoint_mask.astype(jnp.float32).reshape(B, 1, N_NODES)

    grid_spec = pltpu.PrefetchScalarGridSpec(
        num_scalar_prefetch=2,
        grid=(B,),
        in_specs=[
            pl.BlockSpec((1, N_NODES, D_FEAT), lambda b, g, o: (g[b], 0, 0)),
            pl.BlockSpec((1, N_NODES, N_NODES), lambda b, g, o: (g[b], 0, 0)),
            pl.BlockSpec((1, 1, N_NODES), lambda b, g, o: (o[b], 0, 0)),
            pl.BlockSpec((D_FEAT, N_HIDDEN), lambda b, g, o: (0, 0)),
            pl.BlockSpec((1, N_HIDDEN), lambda b, g, o: (0, 0)),
            pl.BlockSpec((N_HIDDEN, N_HIDDEN), lambda b, g, o: (0, 0)),
            pl.BlockSpec((1, N_HIDDEN), lambda b, g, o: (0, 0)),
            pl.BlockSpec((N_HIDDEN, N_HIDDEN), lambda b, g, o: (0, 0)),
            pl.BlockSpec((1, N_HIDDEN), lambda b, g, o: (0, 0)),
            pl.BlockSpec((N_HIDDEN, N_HIDDEN), lambda b, g, o: (0, 0)),
            pl.BlockSpec((1, N_HIDDEN), lambda b, g, o: (0, 0)),
        ],
        out_specs=pl.BlockSpec((1, 1, N_HIDDEN), lambda b, g, o: (o[b], 0, 0)),
        scratch_shapes=[pltpu.VMEM((N_NODES, N_HIDDEN), jnp.float32)],
    )

    out = pl.pallas_call(
        _gcn_kernel,
        grid_spec=grid_spec,
        out_shape=jax.ShapeDtypeStruct((B, 1, N_HIDDEN), jnp.float32),
    )(gids, order, batch_xs, batch_as, mask_f,
      W_in.astype(jnp.bfloat16), b_in.reshape(1, N_HIDDEN),
      W1.astype(jnp.bfloat16), b1.reshape(1, N_HIDDEN),
      W2.astype(jnp.bfloat16), b2.reshape(1, N_HIDDEN),
      W3.astype(jnp.bfloat16), b3.reshape(1, N_HIDDEN))
    return out.reshape(B, N_HIDDEN)


# speedup vs baseline: 1.1669x; 1.1669x over previous
import functools

import jax
import jax.numpy as jnp
from jax.experimental import pallas as pl
from jax.experimental.pallas import tpu as pltpu

N_NODES = 512
D_FEAT = 256
N_HIDDEN = 256


def _dot(a, b):
    return jax.lax.dot_general(
        a, b, (((1,), (0,)), ((), ())),
        precision=jax.lax.Precision.DEFAULT,
        preferred_element_type=jnp.float32)


def _gcn_kernel(gids_ref, order_ref, x_ref, a_ref, mask_ref,
                Win_ref, bin_ref, W1_ref, b1_ref, W2_ref, b2_ref,
                W3_ref, b3_ref, out_ref, h_scratch):
    b = pl.program_id(0)
    prev = gids_ref[jnp.maximum(b - 1, 0)]
    is_new = jnp.logical_or(b == 0, gids_ref[b] != prev)

    @pl.when(is_new)
    def _compute():
        x = x_ref[0]
        a = a_ref[0]
        h0 = jax.nn.relu(_dot(x, Win_ref[...]) + bin_ref[...])
        h = jax.nn.relu(_dot(_dot(a, h0), W1_ref[...]) + b1_ref[...])
        h = jax.nn.relu(_dot(_dot(a, h), W2_ref[...]) + b2_ref[...])
        h = jnp.tanh(_dot(_dot(a, h), W3_ref[...]) + b3_ref[...])
        h_scratch[...] = h + h0

    m = mask_ref[0]
    denom = jnp.maximum(jnp.sum(m), 1.0)
    out_ref[0] = _dot(m, h_scratch[...]) / denom


def kernel(graph, coverpoint_mask, batch_xs, batch_as, W_in, b_in,
           W1, b1, W2, b2, W3, b3):
    B = graph.shape[0]
    order = jnp.argsort(graph).astype(jnp.int32)
    gids = jnp.take(graph, order).astype(jnp.int32)
    mask_f = coverpoint_mask.astype(jnp.float32).reshape(B, 1, N_NODES)

    grid_spec = pltpu.PrefetchScalarGridSpec(
        num_scalar_prefetch=2,
        grid=(B,),
        in_specs=[
            pl.BlockSpec((1, N_NODES, D_FEAT), lambda b, g, o: (g[b], 0, 0)),
            pl.BlockSpec((1, N_NODES, N_NODES), lambda b, g, o: (g[b], 0, 0)),
            pl.BlockSpec((1, 1, N_NODES), lambda b, g, o: (o[b], 0, 0)),
            pl.BlockSpec((D_FEAT, N_HIDDEN), lambda b, g, o: (0, 0)),
            pl.BlockSpec((1, N_HIDDEN), lambda b, g, o: (0, 0)),
            pl.BlockSpec((N_HIDDEN, N_HIDDEN), lambda b, g, o: (0, 0)),
            pl.BlockSpec((1, N_HIDDEN), lambda b, g, o: (0, 0)),
            pl.BlockSpec((N_HIDDEN, N_HIDDEN), lambda b, g, o: (0, 0)),
            pl.BlockSpec((1, N_HIDDEN), lambda b, g, o: (0, 0)),
            pl.BlockSpec((N_HIDDEN, N_HIDDEN), lambda b, g, o: (0, 0)),
            pl.BlockSpec((1, N_HIDDEN), lambda b, g, o: (0, 0)),
        ],
        out_specs=pl.BlockSpec((1, 1, N_HIDDEN), lambda b, g, o: (o[b], 0, 0)),
        scratch_shapes=[pltpu.VMEM((N_NODES, N_HIDDEN), jnp.float32)],
    )

    out = pl.pallas_call(
        _gcn_kernel,
        grid_spec=grid_spec,
        out_shape=jax.ShapeDtypeStruct((B, 1, N_HIDDEN), jnp.float32),
    )(gids, order, batch_xs, batch_as, mask_f,
      W_in, b_in.reshape(1, N_HIDDEN), W1, b1.reshape(1, N_HIDDEN),
      W2, b2.reshape(1, N_HIDDEN), W3, b3.reshape(1, N_HIDDEN))
    return out.reshape(B, N_HIDDEN)
